# Initial kernel scaffold; baseline (speedup 1.0000x reference)
import jax
import jax.numpy as jnp
from jax.experimental import pallas as pl


def kernel(x, edge_index, b_l0_w, b_l0_b, b_l1_w, b_l1_b, b_res_w, b_res_b, m_l0_w, m_l0_b, m_l1_w, m_l1_b, m_res_w, m_res_b):
    raise NotImplementedError("write your pallas kernel here")


# speedup vs baseline: 8.3191x; 8.3191x over previous
import functools

import jax
import jax.numpy as jnp
from jax import lax
from jax.experimental import pallas as pl
from jax.experimental.pallas import tpu as pltpu
from jax.experimental.pallas import tpu_sc as plsc

N = 10000
D = 128
E = 320000
NC = 2
NS = 16
NW = NC * NS
K = 128
CH = -(-E // (NW * K))
EPAD = NW * CH * K
NP = 10240
NROW = NP // NS
NCP = 5
NRC = NROW // NCP
D2 = D // 2

@functools.cache
def _sc_deg_call():
    mesh = plsc.VectorSubcoreMesh(
        core_axis_name="c", subcore_axis_name="s", num_cores=NC, num_subcores=NS)
    return pl.kernel(
        _sc_deg_body,
        compiler_params=pltpu.CompilerParams(use_tc_tiling_on_sc=False),
        out_type=jax.ShapeDtypeStruct((NC * NP, 16), jnp.float32),
        mesh=mesh,
        scratch_types=[
            pltpu.VMEM((CH, K), jnp.int32),
            pltpu.VMEM((K, 16), jnp.float32),
            pltpu.VMEM((NROW, 16), jnp.float32),
            pltpu.VMEM_SHARED((NP, 16), jnp.float32),
        ],
    )


def _sc_deg(*args):
    return _sc_deg_call()(*args)


def _sc_deg_body(col_hbm, ones_hbm, zeros_hbm, out_hbm, col_v, ones_v, stage, acc):
    c = lax.axis_index("c")
    s = lax.axis_index("s")
    w = c * NS + s
    pltpu.sync_copy(col_hbm.at[w], col_v)
    pltpu.sync_copy(ones_hbm, ones_v)
    pltpu.sync_copy(zeros_hbm, stage)
    pltpu.sync_copy(stage, acc.at[pl.ds(s * NROW, NROW)])
    plsc.subcore_barrier()

    @pl.loop(0, CH)
    def _(ch):
        pltpu.sync_copy(ones_v, acc.at[col_v.at[ch]], add=True)

    plsc.subcore_barrier()
    pltpu.sync_copy(acc.at[pl.ds(s * NROW, NROW)], stage)
    pltpu.sync_copy(stage, out_hbm.at[pl.ds(c * NP + s * NROW, NROW)])


@functools.cache
def _sc_scatter_call():
    mesh = plsc.VectorSubcoreMesh(
        core_axis_name="c", subcore_axis_name="s", num_cores=NC, num_subcores=NS)
    return pl.kernel(
        _sc_scatter_body,
        compiler_params=pltpu.CompilerParams(use_tc_tiling_on_sc=False),
        out_type=jax.ShapeDtypeStruct((NC * NP, D2), jnp.float32),
        mesh=mesh,
        scratch_types=[
            pltpu.VMEM((CH, K), jnp.int32),
            pltpu.VMEM((CH, K), jnp.int32),
            pltpu.VMEM((K, D2), jnp.float32),
            pltpu.VMEM((K, D2), jnp.float32),
            pltpu.VMEM_SHARED((NP, D2), jnp.float32),
            pltpu.SemaphoreType.DMA,
            pltpu.SemaphoreType.DMA,
        ],
    )


def _sc_scatter(*args):
    return _sc_scatter_call()(*args)


def _sc_scatter_body(y_hbm, row_hbm, col_hbm, zeros_hbm, out_hbm,
                     row_v, col_v, buf0, buf1, acc, sem0, sem1):
    c = lax.axis_index("c")
    s = lax.axis_index("s")
    w = c * NS + s
    pltpu.sync_copy(row_hbm.at[w], row_v)
    pltpu.sync_copy(col_hbm.at[w], col_v)
    pltpu.sync_copy(zeros_hbm, buf0)
    for kk in range(NCP):
        pltpu.sync_copy(buf0, acc.at[pl.ds(s * NROW + kk * NRC, NRC)])
    plsc.subcore_barrier()

    @pl.loop(0, CH)
    def _(ch):
        pltpu.async_copy(y_hbm.at[row_v.at[ch]], buf0, sem0).wait()
        pltpu.sync_copy(buf0, acc.at[col_v.at[ch]], add=True)

    plsc.subcore_barrier()
    for kk in range(NCP):
        base = s * NROW + kk * NRC
        pltpu.sync_copy(acc.at[pl.ds(base, NRC)], buf0)
        pltpu.sync_copy(buf0, out_hbm.at[pl.ds(c * NP + base, NRC)])


BLK = 1000
_DN = (((1,), (1,)), ((), ()))


def _mlp_block(xb, w0, b0, w1, b1, wr, br):
    res = lax.dot_general(xb, wr, _DN, preferred_element_type=jnp.float32) + br
    h = jnp.maximum(
        lax.dot_general(xb, w0, _DN, preferred_element_type=jnp.float32) + b0, 0.0)
    return lax.dot_general(h, w1, _DN, preferred_element_type=jnp.float32) + b1 + res


def _dis_block(d0, d1):
    deg = d0[:, 0:1] + d1[:, 0:1]
    return lax.rsqrt(deg)


def _stage1_body(d0, d1, x, w0, b0, w1, b1, wr, br, y1a_o, y1b_o, xs0_o):
    dis = _dis_block(d0, d1)
    xb = x[...]
    y1 = dis * xb
    y1a_o[...] = y1[:, :D2]
    y1b_o[...] = y1[:, D2:]
    xs0_o[...] = _mlp_block(xb, w0[...], b0[...], w1[...], b1[...], wr[...], br[...])


def _stage2_body(d0, d1, pa0, pa1, pb0, pb1, w0, b0, w1, b1, wr, br,
                 xs1_o, y2a_o, y2b_o):
    dis = _dis_block(d0, d1)
    cur1 = dis * jnp.concatenate([pa0[...] + pa1[...], pb0[...] + pb1[...]], axis=1)
    xs1_o[...] = _mlp_block(cur1, w0[...], b0[...], w1[...], b1[...], wr[...], br[...])
    y2 = dis * cur1
    y2a_o[...] = y2[:, :D2]
    y2b_o[...] = y2[:, D2:]


def _stage3_body(d0, d1, qa0, qa1, qb0, qb1, xs0, xs1, w0, b0, w1, b1, wr, br,
                 ml0w, ml0b, ml1w, ml1b, mresw, mresb, out_o):
    dis = _dis_block(d0, d1)
    cur2 = dis * jnp.concatenate([qa0[...] + qa1[...], qb0[...] + qb1[...]], axis=1)
    xs2 = _mlp_block(cur2, w0[...], b0[...], w1[...], b1[...], wr[...], br[...])
    h = jnp.concatenate([xs0[...], xs1[...], xs2], axis=1)
    out_o[...] = _mlp_block(h, ml0w[...], ml0b[...], ml1w[...], ml1b[...],
                            mresw[...], mresb[...])


def _row_spec(width):
    return pl.BlockSpec((BLK, width), lambda i: (i, 0))


def _full_spec(shape):
    return pl.BlockSpec(shape, lambda i: (0,) * len(shape))


def _tc_call(body, in_shapes, out_widths):
    grid = (N // BLK,)
    in_specs = []
    for shp in in_shapes:
        if shp[0] == N:
            in_specs.append(_row_spec(shp[1]))
        else:
            in_specs.append(_full_spec(shp))
    return pl.pallas_call(
        body,
        grid=grid,
        in_specs=in_specs,
        out_specs=[_row_spec(w) for w in out_widths],
        out_shape=[jax.ShapeDtypeStruct((N, w), jnp.float32) for w in out_widths],
    )


def kernel(x, edge_index, b_l0_w, b_l0_b, b_l1_w, b_l1_b, b_res_w, b_res_b,
           m_l0_w, m_l0_b, m_l1_w, m_l1_b, m_res_w, m_res_b):
    f32 = jnp.float32
    row = edge_index[0]
    col = edge_index[1]
    pad = EPAD - E
    row_w = jnp.concatenate([row, jnp.full((pad,), N, jnp.int32)]).reshape(NW, CH, K)
    col_w = jnp.concatenate([col, jnp.full((pad,), N, jnp.int32)]).reshape(NW, CH, K)

    ones16 = jnp.ones((K, 16), f32)
    zeros16 = jnp.zeros((NROW, 16), f32)
    zerosD2 = jnp.zeros((K, D2), f32)
    zrow = jnp.zeros((8, D2), f32)

    degp = _sc_deg(col_w, ones16, zeros16)
    d0, d1 = degp[:N], degp[NP:NP + N]

    b1_ = [b.reshape(1, -1) for b in
           (b_l0_b[0], b_l1_b[0], b_res_b[0], b_l0_b[1], b_l1_b[1], b_res_b[1],
            b_l0_b[2], b_l1_b[2], b_res_b[2], m_l0_b, m_l1_b, m_res_b)]
    (bl0b0, bl1b0, brb0, bl0b1, bl1b1, brb1, bl0b2, bl1b2, brb2,
     ml0b, ml1b, mresb) = b1_

    def _gcn_sc(ya, yb):
        pa = _sc_scatter(jnp.concatenate([ya, zrow], axis=0), row_w, col_w, zerosD2)
        pb = _sc_scatter(jnp.concatenate([yb, zrow], axis=0), row_w, col_w, zerosD2)
        return pa[:N], pa[NP:NP + N], pb[:N], pb[NP:NP + N]

    s1_in = [d0, d1, x, b_l0_w[0], bl0b0, b_l1_w[0], bl1b0, b_res_w[0], brb0]
    y1a, y1b, xs0 = _tc_call(_stage1_body, [a.shape for a in s1_in],
                             [D2, D2, D])(*s1_in)

    pa0, pa1, pb0, pb1 = _gcn_sc(y1a, y1b)
    s2_in = [d0, d1, pa0, pa1, pb0, pb1,
             b_l0_w[1], bl0b1, b_l1_w[1], bl1b1, b_res_w[1], brb1]
    xs1, y2a, y2b = _tc_call(_stage2_body, [a.shape for a in s2_in],
                             [D, D2, D2])(*s2_in)

    qa0, qa1, qb0, qb1 = _gcn_sc(y2a, y2b)
    s3_in = [d0, d1, qa0, qa1, qb0, qb1, xs0, xs1, b_l0_w[2], bl0b2, b_l1_w[2],
             bl1b2, b_res_w[2], brb2, m_l0_w, ml0b, m_l1_w, ml1b, m_res_w, mresb]
    (out,) = _tc_call(_stage3_body, [a.shape for a in s3_in], [D])(*s3_in)
    return out


# speedup vs baseline: 10.9938x; 1.3215x over previous
import functools

import jax
import jax.numpy as jnp
from jax import lax
from jax.experimental import pallas as pl
from jax.experimental.pallas import tpu as pltpu
from jax.experimental.pallas import tpu_sc as plsc

N = 10000
D = 128
E = 320000
NC = 2
NS = 16
NW = NC * NS
K = 128
CH = -(-E // (NW * K))
EPAD = NW * CH * K
NP = 10240
NROW = NP // NS
NCP = 5
NRC = NROW // NCP
D2 = D // 2
CHS = EPAD // (NS * K)

@functools.cache
def _sc_deg_call():
    mesh = plsc.VectorSubcoreMesh(
        core_axis_name="c", subcore_axis_name="s", num_cores=NC, num_subcores=NS)
    return pl.kernel(
        _sc_deg_body,
        compiler_params=pltpu.CompilerParams(use_tc_tiling_on_sc=False),
        out_type=jax.ShapeDtypeStruct((NC * NP, 16), jnp.float32),
        mesh=mesh,
        scratch_types=[
            pltpu.VMEM((CH, K), jnp.int32),
            pltpu.VMEM((K, 16), jnp.float32),
            pltpu.VMEM((NROW, 16), jnp.float32),
            pltpu.VMEM_SHARED((NP, 16), jnp.float32),
        ],
    )


def _sc_deg(*args):
    return _sc_deg_call()(*args)


def _sc_deg_body(col_hbm, ones_hbm, zeros_hbm, out_hbm, col_v, ones_v, stage, acc):
    c = lax.axis_index("c")
    s = lax.axis_index("s")
    w = c * NS + s
    pltpu.sync_copy(col_hbm.at[w], col_v)
    pltpu.sync_copy(ones_hbm, ones_v)
    pltpu.sync_copy(zeros_hbm, stage)
    pltpu.sync_copy(stage, acc.at[pl.ds(s * NROW, NROW)])
    plsc.subcore_barrier()

    @pl.loop(0, CH)
    def _(ch):
        pltpu.sync_copy(ones_v, acc.at[col_v.at[ch]], add=True)

    plsc.subcore_barrier()
    pltpu.sync_copy(acc.at[pl.ds(s * NROW, NROW)], stage)
    pltpu.sync_copy(stage, out_hbm.at[pl.ds(c * NP + s * NROW, NROW)])


@functools.cache
def _sc_scatter_call():
    mesh = plsc.VectorSubcoreMesh(
        core_axis_name="c", subcore_axis_name="s", num_cores=NC, num_subcores=NS)
    return pl.kernel(
        _sc_scatter_body,
        compiler_params=pltpu.CompilerParams(use_tc_tiling_on_sc=False),
        out_type=jax.ShapeDtypeStruct((NC * NP, D2), jnp.float32),
        mesh=mesh,
        scratch_types=[
            pltpu.VMEM((CHS, K), jnp.int32),
            pltpu.VMEM((CHS, K), jnp.int32),
            pltpu.VMEM((K, D2), jnp.float32),
            pltpu.VMEM_SHARED((NP, D2), jnp.float32),
            pltpu.SemaphoreType.DMA,
        ],
    )


def _sc_scatter(*args):
    return _sc_scatter_call()(*args)


def _sc_scatter_body(y_hbm, row_hbm, col_hbm, zeros_hbm, out_hbm,
                     row_v, col_v, buf0, acc, sem0):
    c = lax.axis_index("c")
    s = lax.axis_index("s")
    pltpu.sync_copy(row_hbm.at[s], row_v)
    pltpu.sync_copy(col_hbm.at[s], col_v)
    pltpu.sync_copy(zeros_hbm, buf0)
    for kk in range(NCP):
        pltpu.sync_copy(buf0, acc.at[pl.ds(s * NROW + kk * NRC, NRC)])
    plsc.subcore_barrier()

    yc = y_hbm.at[c]
    @pl.loop(0, CHS)
    def _(ch):
        pltpu.async_copy(yc.at[row_v.at[ch]], buf0, sem0).wait()
        pltpu.sync_copy(buf0, acc.at[col_v.at[ch]], add=True)

    plsc.subcore_barrier()
    for kk in range(NCP):
        base = s * NROW + kk * NRC
        pltpu.sync_copy(acc.at[pl.ds(base, NRC)], buf0)
        pltpu.sync_copy(buf0, out_hbm.at[pl.ds(c * NP + base, NRC)])


BLK = 1000
_DN = (((1,), (1,)), ((), ()))


def _mlp_block(xb, w0, b0, w1, b1, wr, br):
    res = lax.dot_general(xb, wr, _DN, preferred_element_type=jnp.float32) + br
    h = jnp.maximum(
        lax.dot_general(xb, w0, _DN, preferred_element_type=jnp.float32) + b0, 0.0)
    return lax.dot_general(h, w1, _DN, preferred_element_type=jnp.float32) + b1 + res


def _dis_block(d0, d1):
    deg = d0[:, 0:1] + d1[:, 0:1]
    return lax.rsqrt(deg)


def _stage1_body(d0, d1, x, w0, b0, w1, b1, wr, br, y1_o, xs0_o):
    dis = _dis_block(d0, d1)
    xb = x[...]
    y1 = dis * xb
    y1_o[0] = y1[:, :D2]
    y1_o[1] = y1[:, D2:]
    xs0_o[...] = _mlp_block(xb, w0[...], b0[...], w1[...], b1[...], wr[...], br[...])


def _stage2_body(d0, d1, pa, pb, w0, b0, w1, b1, wr, br, xs1_o, y2_o):
    dis = _dis_block(d0, d1)
    cur1 = dis * jnp.concatenate([pa[...], pb[...]], axis=1)
    xs1_o[...] = _mlp_block(cur1, w0[...], b0[...], w1[...], b1[...], wr[...], br[...])
    y2 = dis * cur1
    y2_o[0] = y2[:, :D2]
    y2_o[1] = y2[:, D2:]


def _stage3_body(d0, d1, qa, qb, xs0, xs1, w0, b0, w1, b1, wr, br,
                 ml0w, ml0b, ml1w, ml1b, mresw, mresb, out_o):
    dis = _dis_block(d0, d1)
    cur2 = dis * jnp.concatenate([qa[...], qb[...]], axis=1)
    xs2 = _mlp_block(cur2, w0[...], b0[...], w1[...], b1[...], wr[...], br[...])
    h = jnp.concatenate([xs0[...], xs1[...], xs2], axis=1)
    out_o[...] = _mlp_block(h, ml0w[...], ml0b[...], ml1w[...], ml1b[...],
                            mresw[...], mresb[...])


def _row_spec(width):
    return pl.BlockSpec((BLK, width), lambda i: (i, 0))


def _full_spec(shape):
    return pl.BlockSpec(shape, lambda i: (0,) * len(shape))


def _tc_call(body, in_shapes, out_widths):
    grid = (N // BLK,)
    in_specs = []
    for shp in in_shapes:
        if shp[0] == N:
            in_specs.append(_row_spec(shp[1]))
        else:
            in_specs.append(_full_spec(shp))
    out_specs, out_shapes = [], []
    for w in out_widths:
        if isinstance(w, tuple):
            out_specs.append(pl.BlockSpec((2, BLK, w[1]), lambda i: (0, i, 0)))
            out_shapes.append(jax.ShapeDtypeStruct((2, N, w[1]), jnp.float32))
        else:
            out_specs.append(_row_spec(w))
            out_shapes.append(jax.ShapeDtypeStruct((N, w), jnp.float32))
    return pl.pallas_call(
        body,
        grid=grid,
        in_specs=in_specs,
        out_specs=out_specs,
        out_shape=out_shapes,
    )


def kernel(x, edge_index, b_l0_w, b_l0_b, b_l1_w, b_l1_b, b_res_w, b_res_b,
           m_l0_w, m_l0_b, m_l1_w, m_l1_b, m_res_w, m_res_b):
    f32 = jnp.float32
    row = edge_index[0]
    col = edge_index[1]
    pad = EPAD - E
    row_p = jnp.concatenate([row, jnp.zeros((pad,), jnp.int32)])
    col_p = jnp.concatenate([col, jnp.full((pad,), N, jnp.int32)])
    row_w = row_p.reshape(NW, CH, K)
    col_w = col_p.reshape(NW, CH, K)
    row_s = row_p.reshape(NS, CHS, K)
    col_s = col_p.reshape(NS, CHS, K)

    ones16 = jnp.ones((K, 16), f32)
    zeros16 = jnp.zeros((NROW, 16), f32)
    zerosD2 = jnp.zeros((K, D2), f32)

    degp = _sc_deg(col_w, ones16, zeros16)
    d0, d1 = degp[:N], degp[NP:NP + N]

    b1_ = [b.reshape(1, -1) for b in
           (b_l0_b[0], b_l1_b[0], b_res_b[0], b_l0_b[1], b_l1_b[1], b_res_b[1],
            b_l0_b[2], b_l1_b[2], b_res_b[2], m_l0_b, m_l1_b, m_res_b)]
    (bl0b0, bl1b0, brb0, bl0b1, bl1b1, brb1, bl0b2, bl1b2, brb2,
     ml0b, ml1b, mresb) = b1_

    def _gcn_sc(y_st):
        p = _sc_scatter(y_st, row_s, col_s, zerosD2)
        return p[:N], p[NP:NP + N]

    s1_in = [d0, d1, x, b_l0_w[0], bl0b0, b_l1_w[0], bl1b0, b_res_w[0], brb0]
    y1st, xs0 = _tc_call(_stage1_body, [a.shape for a in s1_in],
                         [("stack", D2), D])(*s1_in)

    pa, pb = _gcn_sc(y1st)
    s2_in = [d0, d1, pa, pb, b_l0_w[1], bl0b1, b_l1_w[1], bl1b1, b_res_w[1], brb1]
    xs1, y2st = _tc_call(_stage2_body, [a.shape for a in s2_in],
                         [D, ("stack", D2)])(*s2_in)

    qa, qb = _gcn_sc(y2st)
    s3_in = [d0, d1, qa, qb, xs0, xs1, b_l0_w[2], bl0b2, b_l1_w[2],
             bl1b2, b_res_w[2], brb2, m_l0_w, ml0b, m_l1_w, ml1b, m_res_w, mresb]
    (out,) = _tc_call(_stage3_body, [a.shape for a in s3_in], [D])(*s3_in)
    return out
